# Initial kernel scaffold; baseline (speedup 1.0000x reference)
import jax
import jax.numpy as jnp
from jax.experimental import pallas as pl


def kernel(inputs, embeddings):
    raise NotImplementedError("write your pallas kernel here")


# speedup vs baseline: 1.4958x; 1.4958x over previous
import functools

import jax
import jax.numpy as jnp
from jax import lax
from jax.experimental import pallas as pl
from jax.experimental.pallas import tpu as pltpu
from jax.experimental.pallas import tpu_sc as plsc

NUM_E = 8192
DIM = 32
B, C, L = 8, 32, 2048
NTOK = B * L
TL = 512
CT = 1024
HALF = NUM_E // 2
NJ = L // TL
COMMIT = 0.25


def _argmin_body(xt_ref, e_ref, idx_ref, loss_ref):
    xt = xt_ref[0]
    xb = xt.astype(jnp.bfloat16)
    x2 = jnp.sum(xt * xt, axis=0, keepdims=True)

    def half_argmin(h):
        run_min = jnp.full((1, TL), jnp.inf, jnp.float32)
        run_idx = jnp.zeros((1, TL), jnp.int32)
        for k in range(h * (HALF // CT), (h + 1) * (HALF // CT)):
            e = e_ref[k * CT:(k + 1) * CT, :]
            e2 = jnp.sum(e * e, axis=1, keepdims=True)
            mm = lax.dot_general(
                e, xb, (((1,), (0,)), ((), ())),
                preferred_element_type=jnp.float32)
            d = (x2 + e2) - 2.0 * mm
            dmin = jnp.min(d, axis=0, keepdims=True)
            row = lax.broadcasted_iota(jnp.int32, (CT, TL), 0) + (k * CT)
            lidx = jnp.min(jnp.where(d == dmin, row, jnp.int32(2 ** 30)),
                           axis=0, keepdims=True)
            better = dmin < run_min
            run_idx = jnp.where(better, lidx, run_idx)
            run_min = jnp.where(better, dmin, run_min)
        return run_min, run_idx

    m0, i0 = half_argmin(0)
    m1, i1 = half_argmin(1)
    m0r = m0.astype(jnp.bfloat16).astype(jnp.float32)
    keep0 = m0r <= m1
    idx_ref[0] = jnp.where(keep0, i0, i1)
    chosen = jnp.where(keep0, m0, m1)

    first = jnp.logical_and(pl.program_id(0) == 0, pl.program_id(1) == 0)
    last = jnp.logical_and(pl.program_id(0) == B - 1,
                           pl.program_id(1) == NJ - 1)

    @pl.when(first)
    def _init():
        loss_ref[0, 0] = 0.0

    loss_ref[0, 0] += jnp.sum(chosen)

    @pl.when(last)
    def _fini():
        loss_ref[0, 0] = loss_ref[0, 0] * ((1.0 + COMMIT) / (NTOK * DIM))


_argmin_call = pl.pallas_call(
    _argmin_body,
    grid=(B, NJ),
    in_specs=[
        pl.BlockSpec((1, DIM, TL), lambda b, j: (b, 0, j)),
        pl.BlockSpec((NUM_E, DIM), lambda b, j: (0, 0)),
    ],
    out_specs=[
        pl.BlockSpec((1, 1, TL), lambda b, j: (b, 0, j)),
        pl.BlockSpec(block_shape=(1, 1), index_map=lambda b, j: (0, 0),
                     memory_space=pltpu.SMEM),
    ],
    out_shape=[
        jax.ShapeDtypeStruct((B, 1, L), jnp.int32),
        jax.ShapeDtypeStruct((1, 1), jnp.float32),
    ],
)


_NC, _NS = 2, 16
_NW = _NC * _NS
_BPW = NTOK // _NW


def _gather_body(table_hbm, idx_hbm, out_hbm, idx_v, rows_v, sem):
    wid = lax.axis_index("s") * _NC + lax.axis_index("c")
    base = wid * _BPW
    pltpu.sync_copy(idx_hbm.at[pl.ds(base, _BPW)], idx_v)
    pltpu.async_copy(table_hbm.at[idx_v], rows_v, sem).wait()
    pltpu.sync_copy(rows_v, out_hbm.at[pl.ds(base, _BPW)])


@functools.lru_cache(maxsize=1)
def _gather_rows():
    return pl.kernel(
        _gather_body,
        out_type=jax.ShapeDtypeStruct((NTOK, DIM), jnp.float32),
        mesh=plsc.VectorSubcoreMesh(core_axis_name="c", subcore_axis_name="s",
                                    num_cores=_NC, num_subcores=_NS),
        scratch_types=[
            pltpu.VMEM((_BPW,), jnp.int32),
            pltpu.VMEM((_BPW, DIM), jnp.float32),
            pltpu.SemaphoreType.DMA,
        ],
        compiler_params=pltpu.CompilerParams(use_tc_tiling_on_sc=False),
    )


def kernel(inputs, embeddings):
    idx3, loss = _argmin_call(inputs, embeddings)
    idx = idx3.reshape(NTOK)
    q_flat = _gather_rows()(embeddings, idx)
    q = q_flat.reshape(B, L, DIM)
    x = jnp.transpose(inputs, (0, 2, 1))
    q = x + (q - x)
    quantized = jnp.transpose(q, (0, 2, 1))
    return quantized, loss[0, 0], idx3.reshape(B, L)


# speedup vs baseline: 2.1645x; 1.4470x over previous
import functools

import jax
import jax.numpy as jnp
from jax import lax
from jax.experimental import pallas as pl
from jax.experimental.pallas import tpu as pltpu
from jax.experimental.pallas import tpu_sc as plsc

NUM_E = 8192
DIM = 32
B, C, L = 8, 32, 2048
NTOK = B * L
TL = 512
CT = 1024
HALF = NUM_E // 2
NJ = L // TL
COMMIT = 0.25


def _argmin_body(xt_ref, e_ref, idx_ref, loss_ref):
    xt = xt_ref[0]
    xb = xt.astype(jnp.bfloat16)
    x2 = jnp.sum(xt * xt, axis=0, keepdims=True)

    sub = lax.broadcasted_iota(jnp.int32, (8, TL), 0).astype(jnp.float32)

    def half_argmin(h):
        run_v = jnp.full((8, TL), jnp.inf, jnp.float32)
        run_r = jnp.zeros((8, TL), jnp.float32)
        for k in range(h * (HALF // CT), (h + 1) * (HALF // CT)):
            e = e_ref[k * CT:(k + 1) * CT, :]
            e2 = jnp.sum(e * e, axis=1, keepdims=True)
            mm2 = lax.dot_general(
                e * (-2.0), xb, (((1,), (0,)), ((), ())),
                preferred_element_type=jnp.float32)
            d3 = ((x2 + e2) + mm2).reshape(CT // 8, 8, TL)
            for r in range(CT // 8):
                dr = d3[r]
                lt = dr < run_v
                run_v = jnp.minimum(run_v, dr)
                run_r = jnp.where(lt, jnp.float32((k - h * (HALF // CT))
                                                  * (CT // 8) + r), run_r)
        gmin = jnp.min(run_v, axis=0, keepdims=True)
        code = run_r * jnp.float32(8.0) + sub
        cand = jnp.where(run_v == gmin, code, jnp.float32(2 ** 30))
        gidx = jnp.min(cand, axis=0, keepdims=True) + jnp.float32(h * HALF)
        return gmin, gidx

    m0, i0 = half_argmin(0)
    m1, i1 = half_argmin(1)
    m0r = m0.astype(jnp.bfloat16).astype(jnp.float32)
    keep0 = m0r <= m1
    idx_ref[0] = jnp.where(keep0, i0, i1).astype(jnp.int32)
    chosen = jnp.where(keep0, m0, m1)

    first = jnp.logical_and(pl.program_id(0) == 0, pl.program_id(1) == 0)
    last = jnp.logical_and(pl.program_id(0) == B - 1,
                           pl.program_id(1) == NJ - 1)

    @pl.when(first)
    def _init():
        loss_ref[0, 0] = 0.0

    loss_ref[0, 0] += jnp.sum(chosen)

    @pl.when(last)
    def _fini():
        loss_ref[0, 0] = loss_ref[0, 0] * ((1.0 + COMMIT) / (NTOK * DIM))


_argmin_call = pl.pallas_call(
    _argmin_body,
    grid=(B, NJ),
    in_specs=[
        pl.BlockSpec((1, DIM, TL), lambda b, j: (b, 0, j)),
        pl.BlockSpec((NUM_E, DIM), lambda b, j: (0, 0)),
    ],
    out_specs=[
        pl.BlockSpec((1, 1, TL), lambda b, j: (b, 0, j)),
        pl.BlockSpec(block_shape=(1, 1), index_map=lambda b, j: (0, 0),
                     memory_space=pltpu.SMEM),
    ],
    out_shape=[
        jax.ShapeDtypeStruct((B, 1, L), jnp.int32),
        jax.ShapeDtypeStruct((1, 1), jnp.float32),
    ],
)


_NC, _NS = 2, 16
_NW = _NC * _NS
_BPW = NTOK // _NW


def _gather_body(table_hbm, idx_hbm, out_hbm, idx_v, rows_v, sem):
    wid = lax.axis_index("s") * _NC + lax.axis_index("c")
    base = wid * _BPW
    pltpu.sync_copy(idx_hbm.at[pl.ds(base, _BPW)], idx_v)
    pltpu.async_copy(table_hbm.at[idx_v], rows_v, sem).wait()
    pltpu.sync_copy(rows_v, out_hbm.at[pl.ds(base, _BPW)])


@functools.lru_cache(maxsize=1)
def _gather_rows():
    return pl.kernel(
        _gather_body,
        out_type=jax.ShapeDtypeStruct((NTOK, DIM), jnp.float32),
        mesh=plsc.VectorSubcoreMesh(core_axis_name="c", subcore_axis_name="s",
                                    num_cores=_NC, num_subcores=_NS),
        scratch_types=[
            pltpu.VMEM((_BPW,), jnp.int32),
            pltpu.VMEM((_BPW, DIM), jnp.float32),
            pltpu.SemaphoreType.DMA,
        ],
        compiler_params=pltpu.CompilerParams(use_tc_tiling_on_sc=False),
    )


def kernel(inputs, embeddings):
    idx3, loss = _argmin_call(inputs, embeddings)
    idx = idx3.reshape(NTOK)
    q_flat = _gather_rows()(embeddings, idx)
    q = q_flat.reshape(B, L, DIM)
    x = jnp.transpose(inputs, (0, 2, 1))
    q = x + (q - x)
    quantized = jnp.transpose(q, (0, 2, 1))
    return quantized, loss[0, 0], idx3.reshape(B, L)


# speedup vs baseline: 2.3936x; 1.1059x over previous
import functools

import jax
import jax.numpy as jnp
from jax import lax
from jax.experimental import pallas as pl
from jax.experimental.pallas import tpu as pltpu
from jax.experimental.pallas import tpu_sc as plsc

NUM_E = 8192
DIM = 32
B, C, L = 8, 32, 2048
NTOK = B * L
TL = 2048
CT = 1024
HALF = NUM_E // 2
NJ = L // TL
COMMIT = 0.25


def _argmin_body(xt_ref, e_ref, idx_ref, loss_ref):
    xt = xt_ref[0]
    xb = xt.astype(jnp.bfloat16)
    x2 = jnp.sum(xt * xt, axis=0, keepdims=True)

    sub = lax.broadcasted_iota(jnp.int32, (8, TL), 0).astype(jnp.float32)

    def half_argmin(h):
        run_v = jnp.full((8, TL), jnp.inf, jnp.float32)
        run_r = jnp.zeros((8, TL), jnp.float32)
        for k in range(h * (HALF // CT), (h + 1) * (HALF // CT)):
            e = e_ref[k * CT:(k + 1) * CT, :]
            e2 = jnp.sum(e * e, axis=1, keepdims=True)
            mm2 = lax.dot_general(
                e * (-2.0), xb, (((1,), (0,)), ((), ())),
                preferred_element_type=jnp.float32)
            d3 = ((x2 + e2) + mm2).reshape(CT // 8, 8, TL)
            for r in range(CT // 8):
                dr = d3[r]
                lt = dr < run_v
                run_v = jnp.minimum(run_v, dr)
                run_r = jnp.where(lt, jnp.float32((k - h * (HALF // CT))
                                                  * (CT // 8) + r), run_r)
        gmin = jnp.min(run_v, axis=0, keepdims=True)
        code = run_r * jnp.float32(8.0) + sub
        cand = jnp.where(run_v == gmin, code, jnp.float32(2 ** 30))
        gidx = jnp.min(cand, axis=0, keepdims=True) + jnp.float32(h * HALF)
        return gmin, gidx

    m0, i0 = half_argmin(0)
    m1, i1 = half_argmin(1)
    m0r = m0.astype(jnp.bfloat16).astype(jnp.float32)
    keep0 = m0r <= m1
    idx_ref[0] = jnp.where(keep0, i0, i1).astype(jnp.int32)
    chosen = jnp.where(keep0, m0, m1)

    first = jnp.logical_and(pl.program_id(0) == 0, pl.program_id(1) == 0)
    last = jnp.logical_and(pl.program_id(0) == B - 1,
                           pl.program_id(1) == NJ - 1)

    @pl.when(first)
    def _init():
        loss_ref[0, 0] = 0.0

    loss_ref[0, 0] += jnp.sum(chosen)

    @pl.when(last)
    def _fini():
        loss_ref[0, 0] = loss_ref[0, 0] * ((1.0 + COMMIT) / (NTOK * DIM))


_argmin_call = pl.pallas_call(
    _argmin_body,
    grid=(B, NJ),
    in_specs=[
        pl.BlockSpec((1, DIM, TL), lambda b, j: (b, 0, j)),
        pl.BlockSpec((NUM_E, DIM), lambda b, j: (0, 0)),
    ],
    out_specs=[
        pl.BlockSpec((1, 1, TL), lambda b, j: (b, 0, j)),
        pl.BlockSpec(block_shape=(1, 1), index_map=lambda b, j: (0, 0),
                     memory_space=pltpu.SMEM),
    ],
    out_shape=[
        jax.ShapeDtypeStruct((B, 1, L), jnp.int32),
        jax.ShapeDtypeStruct((1, 1), jnp.float32),
    ],
)


_NC, _NS = 2, 16
_NW = _NC * _NS
_BPW = NTOK // _NW


def _gather_body(table_hbm, idx_hbm, out_hbm, idx_v, rows_v, sem):
    wid = lax.axis_index("s") * _NC + lax.axis_index("c")
    base = wid * _BPW
    pltpu.sync_copy(idx_hbm.at[pl.ds(base, _BPW)], idx_v)
    pltpu.async_copy(table_hbm.at[idx_v], rows_v, sem).wait()
    pltpu.sync_copy(rows_v, out_hbm.at[pl.ds(base, _BPW)])


@functools.lru_cache(maxsize=1)
def _gather_rows():
    return pl.kernel(
        _gather_body,
        out_type=jax.ShapeDtypeStruct((NTOK, DIM), jnp.float32),
        mesh=plsc.VectorSubcoreMesh(core_axis_name="c", subcore_axis_name="s",
                                    num_cores=_NC, num_subcores=_NS),
        scratch_types=[
            pltpu.VMEM((_BPW,), jnp.int32),
            pltpu.VMEM((_BPW, DIM), jnp.float32),
            pltpu.SemaphoreType.DMA,
        ],
        compiler_params=pltpu.CompilerParams(use_tc_tiling_on_sc=False),
    )


def kernel(inputs, embeddings):
    idx3, loss = _argmin_call(inputs, embeddings)
    idx = idx3.reshape(NTOK)
    q_flat = _gather_rows()(embeddings, idx)
    q = q_flat.reshape(B, L, DIM)
    x = jnp.transpose(inputs, (0, 2, 1))
    q = x + (q - x)
    quantized = jnp.transpose(q, (0, 2, 1))
    return quantized, loss[0, 0], idx3.reshape(B, L)


# speedup vs baseline: 2.3993x; 1.0024x over previous
import functools

import jax
import jax.numpy as jnp
from jax import lax
from jax.experimental import pallas as pl
from jax.experimental.pallas import tpu as pltpu
from jax.experimental.pallas import tpu_sc as plsc

NUM_E = 8192
DIM = 32
B, C, L = 8, 32, 2048
NTOK = B * L
TL = 2048
CT = 1024
HALF = NUM_E // 2
NJ = L // TL
COMMIT = 0.25


def _argmin_body(xt_ref, e_ref, idx_ref, loss_ref):
    xt = xt_ref[0]
    xb = xt.astype(jnp.bfloat16)
    x2 = jnp.sum(xt * xt, axis=0, keepdims=True)

    sub = lax.broadcasted_iota(jnp.int32, (8, TL), 0).astype(jnp.float32)

    def half_argmin(h):
        run_v = jnp.full((8, TL), jnp.inf, jnp.float32)
        run_r = jnp.zeros((8, TL), jnp.float32)
        for k in range(h * (HALF // CT), (h + 1) * (HALF // CT)):
            e = e_ref[k * CT:(k + 1) * CT, :]
            e2 = jnp.sum(e * e, axis=1, keepdims=True)
            mm2 = lax.dot_general(
                e * (-2.0), xb, (((1,), (0,)), ((), ())),
                preferred_element_type=jnp.float32)
            d3 = ((x2 + e2) + mm2).reshape(CT // 8, 8, TL)
            for r in range(CT // 8):
                dr = d3[r]
                lt = dr < run_v
                run_v = jnp.minimum(run_v, dr)
                run_r = jnp.where(lt, jnp.float32((k - h * (HALF // CT))
                                                  * (CT // 8) + r), run_r)
        gmin = jnp.min(run_v, axis=0, keepdims=True)
        code = run_r * jnp.float32(8.0) + sub
        cand = jnp.where(run_v == gmin, code, jnp.float32(2 ** 30))
        gidx = jnp.min(cand, axis=0, keepdims=True) + jnp.float32(h * HALF)
        return gmin, gidx

    m0, i0 = half_argmin(0)
    m1, i1 = half_argmin(1)
    m0r = m0.astype(jnp.bfloat16).astype(jnp.float32)
    keep0 = m0r <= m1
    idx_ref[0] = jnp.where(keep0, i0, i1).astype(jnp.int32)
    chosen = jnp.where(keep0, m0, m1)

    first = jnp.logical_and(pl.program_id(0) == 0, pl.program_id(1) == 0)
    last = jnp.logical_and(pl.program_id(0) == B - 1,
                           pl.program_id(1) == NJ - 1)

    @pl.when(first)
    def _init():
        loss_ref[0, 0] = 0.0

    loss_ref[0, 0] += jnp.sum(chosen)

    @pl.when(last)
    def _fini():
        loss_ref[0, 0] = loss_ref[0, 0] * ((1.0 + COMMIT) / (NTOK * DIM))


_argmin_call = pl.pallas_call(
    _argmin_body,
    grid=(B, NJ),
    in_specs=[
        pl.BlockSpec((1, DIM, TL), lambda b, j: (b, 0, j)),
        pl.BlockSpec((NUM_E, DIM), lambda b, j: (0, 0)),
    ],
    out_specs=[
        pl.BlockSpec((1, 1, TL), lambda b, j: (b, 0, j)),
        pl.BlockSpec(block_shape=(1, 1), index_map=lambda b, j: (0, 0),
                     memory_space=pltpu.SMEM),
    ],
    out_shape=[
        jax.ShapeDtypeStruct((B, 1, L), jnp.int32),
        jax.ShapeDtypeStruct((1, 1), jnp.float32),
    ],
)


_NC, _NS = 2, 16
_NW = _NC * _NS
_BPW = NTOK // _NW


def _gather_body(table_hbm, idx_hbm, out_hbm, idx_v, rows_v, sem):
    wid = lax.axis_index("s") * _NC + lax.axis_index("c")
    base = wid * _BPW
    pltpu.sync_copy(idx_hbm.at[pl.ds(base, _BPW)], idx_v)
    pltpu.async_copy(table_hbm.at[idx_v], rows_v, sem).wait()
    pltpu.sync_copy(rows_v, out_hbm.at[pl.ds(base, _BPW)])


@functools.lru_cache(maxsize=1)
def _gather_rows():
    return pl.kernel(
        _gather_body,
        out_type=jax.ShapeDtypeStruct((NTOK, DIM), jnp.float32),
        mesh=plsc.VectorSubcoreMesh(core_axis_name="c", subcore_axis_name="s",
                                    num_cores=_NC, num_subcores=_NS),
        scratch_types=[
            pltpu.VMEM((_BPW,), jnp.int32),
            pltpu.VMEM((_BPW, DIM), jnp.float32),
            pltpu.SemaphoreType.DMA,
        ],
        compiler_params=pltpu.CompilerParams(use_tc_tiling_on_sc=False),
    )


def kernel(inputs, embeddings):
    idx3, loss = _argmin_call(inputs, embeddings)
    idx = idx3.reshape(NTOK)
    q_flat = _gather_rows()(embeddings, idx)
    quantized = jnp.transpose(q_flat.reshape(B, L, DIM), (0, 2, 1))
    return quantized, loss[0, 0], idx3.reshape(B, L)


# speedup vs baseline: 2.4248x; 1.0106x over previous
import functools

import jax
import jax.numpy as jnp
from jax import lax
from jax.experimental import pallas as pl
from jax.experimental.pallas import tpu as pltpu
from jax.experimental.pallas import tpu_sc as plsc

NUM_E = 8192
DIM = 32
B, C, L = 8, 32, 2048
NTOK = B * L
TL = 2048
CT = 2048
HALF = NUM_E // 2
NJ = L // TL
COMMIT = 0.25


def _argmin_body(xt_ref, e_ref, idx_ref, loss_ref):
    xt = xt_ref[0]
    xb = xt.astype(jnp.bfloat16)
    x2 = jnp.sum(xt * xt, axis=0, keepdims=True)

    sub = lax.broadcasted_iota(jnp.int32, (8, TL), 0).astype(jnp.float32)

    def half_argmin(h):
        run_v = jnp.full((8, TL), jnp.inf, jnp.float32)
        run_r = jnp.zeros((8, TL), jnp.float32)
        for k in range(h * (HALF // CT), (h + 1) * (HALF // CT)):
            e = e_ref[k * CT:(k + 1) * CT, :]
            e2 = jnp.sum(e * e, axis=1, keepdims=True)
            mm2 = lax.dot_general(
                e * (-2.0), xb, (((1,), (0,)), ((), ())),
                preferred_element_type=jnp.float32)
            d3 = ((x2 + e2) + mm2).reshape(CT // 8, 8, TL)
            for r in range(CT // 8):
                dr = d3[r]
                lt = dr < run_v
                run_v = jnp.minimum(run_v, dr)
                run_r = jnp.where(lt, jnp.float32((k - h * (HALF // CT))
                                                  * (CT // 8) + r), run_r)
        gmin = jnp.min(run_v, axis=0, keepdims=True)
        code = run_r * jnp.float32(8.0) + sub
        cand = jnp.where(run_v == gmin, code, jnp.float32(2 ** 30))
        gidx = jnp.min(cand, axis=0, keepdims=True) + jnp.float32(h * HALF)
        return gmin, gidx

    m0, i0 = half_argmin(0)
    m1, i1 = half_argmin(1)
    m0r = m0.astype(jnp.bfloat16).astype(jnp.float32)
    keep0 = m0r <= m1
    idx_ref[0] = jnp.where(keep0, i0, i1).astype(jnp.int32)
    chosen = jnp.where(keep0, m0, m1)

    first = jnp.logical_and(pl.program_id(0) == 0, pl.program_id(1) == 0)
    last = jnp.logical_and(pl.program_id(0) == B - 1,
                           pl.program_id(1) == NJ - 1)

    @pl.when(first)
    def _init():
        loss_ref[0, 0] = 0.0

    loss_ref[0, 0] += jnp.sum(chosen)

    @pl.when(last)
    def _fini():
        loss_ref[0, 0] = loss_ref[0, 0] * ((1.0 + COMMIT) / (NTOK * DIM))


_argmin_call = pl.pallas_call(
    _argmin_body,
    grid=(B, NJ),
    in_specs=[
        pl.BlockSpec((1, DIM, TL), lambda b, j: (b, 0, j)),
        pl.BlockSpec((NUM_E, DIM), lambda b, j: (0, 0)),
    ],
    out_specs=[
        pl.BlockSpec((1, 1, TL), lambda b, j: (b, 0, j)),
        pl.BlockSpec(block_shape=(1, 1), index_map=lambda b, j: (0, 0),
                     memory_space=pltpu.SMEM),
    ],
    out_shape=[
        jax.ShapeDtypeStruct((B, 1, L), jnp.int32),
        jax.ShapeDtypeStruct((1, 1), jnp.float32),
    ],
)


_NC, _NS = 2, 16
_NW = _NC * _NS
_BPW = NTOK // _NW


def _gather_body(table_hbm, idx_hbm, out_hbm, idx_v, rows_v, sem):
    wid = lax.axis_index("s") * _NC + lax.axis_index("c")
    base = wid * _BPW
    pltpu.sync_copy(idx_hbm.at[pl.ds(base, _BPW)], idx_v)
    pltpu.async_copy(table_hbm.at[idx_v], rows_v, sem).wait()
    pltpu.sync_copy(rows_v, out_hbm.at[pl.ds(base, _BPW)])


@functools.lru_cache(maxsize=1)
def _gather_rows():
    return pl.kernel(
        _gather_body,
        out_type=jax.ShapeDtypeStruct((NTOK, DIM), jnp.float32),
        mesh=plsc.VectorSubcoreMesh(core_axis_name="c", subcore_axis_name="s",
                                    num_cores=_NC, num_subcores=_NS),
        scratch_types=[
            pltpu.VMEM((_BPW,), jnp.int32),
            pltpu.VMEM((_BPW, DIM), jnp.float32),
            pltpu.SemaphoreType.DMA,
        ],
        compiler_params=pltpu.CompilerParams(use_tc_tiling_on_sc=False),
    )


def kernel(inputs, embeddings):
    idx3, loss = _argmin_call(inputs, embeddings)
    idx = idx3.reshape(NTOK)
    q_flat = _gather_rows()(embeddings, idx)
    quantized = jnp.transpose(q_flat.reshape(B, L, DIM), (0, 2, 1))
    return quantized, loss[0, 0], idx3.reshape(B, L)
